# baseline (speedup 1.0000x reference)
import functools

import jax
import jax.numpy as jnp
from jax import lax
from jax.experimental import pallas as pl
from jax.experimental.pallas import tpu as pltpu
from jax.experimental.pallas import tpu_sc as plsc

_NCODES = 16384
_BATCH = 4096
_D = 512
_L = 16
_NC = 2
_NS = 16
_NW = _NC * _NS
_BPW = _BATCH // _NW
_FCH = _BPW // _L
_NCHUNKS = _BATCH // _L


def _body(idx_hbm, data_hbm, out_hbm, idx_v, table_v, fsrc_v, fdst_v,
          rows_v, csem, psem):
    wid = lax.axis_index("s") * _NC + lax.axis_index("c")
    base = wid * _BPW

    lin_in = pltpu.async_copy(
        data_hbm.at[pl.ds(base, _BPW)], rows_v, csem)

    pltpu.sync_copy(idx_hbm, idx_v)

    lane = lax.iota(jnp.int32, _L)
    nxt_lane = (lane + 1) & (_L - 1)
    last_lane = lane == (_L - 1)

    def chunk_step(c, carry):
        chunk = idx_v[pl.ds(c * _L, _L)]
        comp = chunk * _L + lane
        sk, _ = plsc.sort_key_val(comp, comp)
        nxt = jnp.take(sk, nxt_lane, mode="wrap")
        code = sk >> 4
        is_last = jnp.logical_or(code != (nxt >> 4), last_lane)
        pos = (sk & (_L - 1)) + c * _L
        plsc.store_scatter(table_v, [code], pos, mask=is_last)
        return carry

    lax.fori_loop(0, _NCHUNKS, chunk_step, 0, unroll=8)

    n = jnp.int32(0)
    for b in range(_FCH):
        my = idx_v[pl.ds(base + b * _L, _L)]
        s = plsc.load_gather(table_v, [my])
        rows = base + b * _L + lane
        m = s != rows
        mi = m.astype(jnp.int32)
        posn = n + jnp.cumsum(mi) - 1
        plsc.store_scatter(fsrc_v, [posn], s, mask=m)
        plsc.store_scatter(fdst_v, [posn], rows, mask=m)
        n = n + jnp.sum(mi)

    lin_in.wait()
    zero = jnp.zeros((_L,), jnp.int32)

    def patch_step(r, carry):
        br = zero + r
        ss = jnp.max(plsc.load_gather(fsrc_v, [br]))
        dd = jnp.max(plsc.load_gather(fdst_v, [br])) - base
        pltpu.async_copy(
            data_hbm.at[pl.ds(ss, 1)], rows_v.at[pl.ds(dd, 1)], psem)
        return carry

    lax.fori_loop(0, n, patch_step, 0)

    def drain_step(r, carry):
        pltpu.make_async_copy(
            data_hbm.at[pl.ds(0, 1)], rows_v.at[pl.ds(0, 1)], psem).wait()
        return carry

    lax.fori_loop(0, n, drain_step, 0)

    pltpu.sync_copy(rows_v, out_hbm.at[pl.ds(base, _BPW)])


_cache_lookup = functools.partial(
    pl.kernel,
    out_type=jax.ShapeDtypeStruct((_BATCH, _D), jnp.float32),
    mesh=plsc.VectorSubcoreMesh(
        core_axis_name="c", subcore_axis_name="s",
        num_cores=_NC, num_subcores=_NS),
    scratch_types=[
        pltpu.VMEM((_BATCH,), jnp.int32),
        pltpu.VMEM((_NCODES,), jnp.int32),
        pltpu.VMEM((_BPW,), jnp.int32),
        pltpu.VMEM((_BPW,), jnp.int32),
        pltpu.VMEM((_BPW, _D), jnp.float32),
        pltpu.SemaphoreType.DMA,
        pltpu.SemaphoreType.DMA,
    ],
    compiler_params=pltpu.CompilerParams(needs_layout_passes=False),
)(_body)


@jax.jit
def kernel(seq_idxs, set_data, cache):
    del cache
    return _cache_lookup(seq_idxs.astype(jnp.int32), set_data)


# speedup vs baseline: 1.0130x; 1.0130x over previous
import functools

import jax
import jax.numpy as jnp
from jax import lax
from jax.experimental import pallas as pl
from jax.experimental.pallas import tpu as pltpu
from jax.experimental.pallas import tpu_sc as plsc

_NCODES = 16384
_BATCH = 4096
_D = 512
_L = 16
_NC = 2
_NS = 16
_NW = _NC * _NS
_BPW = _BATCH // _NW
_NCHUNKS = _BATCH // _L


def _body(idx_hbm, data_hbm, out_hbm, idx_v, table_v, src_v, rows_v, sem):
    wid = lax.axis_index("s") * _NC + lax.axis_index("c")

    pltpu.sync_copy(idx_hbm, idx_v)

    lane = lax.iota(jnp.int32, _L)
    nxt_lane = (lane + 1) & (_L - 1)
    last_lane = lane == (_L - 1)

    def chunk_step(c, carry):
        chunk = idx_v[pl.ds(c * _L, _L)]
        comp = chunk * _L + lane
        sk, _ = plsc.sort_key_val(comp, comp)
        nxt = jnp.take(sk, nxt_lane, mode="wrap")
        code = sk >> 4
        is_last = jnp.logical_or(code != (nxt >> 4), last_lane)
        pos = (sk & (_L - 1)) + c * _L
        plsc.store_scatter(table_v, [code], pos, mask=is_last)
        return carry

    lax.fori_loop(0, _NCHUNKS, chunk_step, 0, unroll=8)

    base = wid * _BPW
    for b in range(_BPW // _L):
        my = idx_v[pl.ds(base + b * _L, _L)]
        src_v[pl.ds(b * _L, _L)] = plsc.load_gather(table_v, [my])

    pltpu.async_copy(data_hbm.at[src_v], rows_v, sem).wait()
    pltpu.sync_copy(rows_v, out_hbm.at[pl.ds(base, _BPW)])


_cache_lookup = functools.partial(
    pl.kernel,
    out_type=jax.ShapeDtypeStruct((_BATCH, _D), jnp.float32),
    mesh=plsc.VectorSubcoreMesh(
        core_axis_name="c", subcore_axis_name="s",
        num_cores=_NC, num_subcores=_NS),
    scratch_types=[
        pltpu.VMEM((_BATCH,), jnp.int32),
        pltpu.VMEM((_NCODES,), jnp.int32),
        pltpu.VMEM((_BPW,), jnp.int32),
        pltpu.VMEM((_BPW, _D), jnp.float32),
        pltpu.SemaphoreType.DMA,
    ],
    compiler_params=pltpu.CompilerParams(needs_layout_passes=False),
)(_body)


@jax.jit
def kernel(seq_idxs, set_data, cache):
    del cache
    return _cache_lookup(seq_idxs.astype(jnp.int32), set_data)
